# baseline (speedup 1.0000x reference)
import functools

import jax
import jax.numpy as jnp
from jax import lax
from jax.experimental import pallas as pl
from jax.experimental.pallas import tpu as pltpu
from jax.experimental.pallas import tpu_sc as plsc

EPS = 1e-10
NUM_CORES = 2
NUM_SUBCORES = 16
LANES = 16
NW = NUM_CORES * NUM_SUBCORES
TILE = 128
KPT = TILE // LANES
NEG_INF = float("-inf")
INT_MAX = 2147483647

def _tc_log_noise(exponential):
    V = exponential.shape[1]
    C = 131072
    grid = -(-V // C)

    def body(e_ref, n_ref):
        n_ref[...] = jnp.log(jnp.maximum(e_ref[0, :], EPS))

    return pl.pallas_call(
        body,
        grid=(grid,),
        in_specs=[pl.BlockSpec((1, C), lambda i: (0, i))],
        out_specs=pl.BlockSpec((C,), lambda i: (i,)),
        out_shape=jax.ShapeDtypeStruct((V,), jnp.float32),
    )(exponential)


def _merge(cand):
    while len(cand) > 1:
        nxt = []
        for i in range(0, len(cand) - 1, 2):
            av, ai = cand[i]
            bv, bi = cand[i + 1]
            takeb = (bv > av) | ((bv == av) & (bi < ai))
            nxt.append((jnp.where(takeb, bv, av), jnp.where(takeb, bi, ai)))
        if len(cand) % 2:
            nxt.append(cand[-1])
        cand = nxt
    return cand[0]


@functools.lru_cache(maxsize=None)
def _build_sc_sampler(B, V):
    assert B == 32
    NT = V // TILE
    TPW = -(-NT // NW)
    CT = 36
    TPW = -(-TPW // CT) * CT
    NCH = TPW // CT
    assert NCH >= 3 and TPW <= NT
    STEP = -(-(NT - TPW) // (NW - 1))
    assert STEP <= TPW and (NW - 2) * STEP + TPW >= NT - TPW

    mesh = plsc.VectorSubcoreMesh(
        core_axis_name="c", subcore_axis_name="s",
        num_cores=NUM_CORES, num_subcores=NUM_SUBCORES)

    @functools.partial(
        pl.kernel,
        mesh=mesh,
        compiler_params=pltpu.CompilerParams(
            needs_layout_passes=False, use_tc_tiling_on_sc=True),
        out_type=[
            jax.ShapeDtypeStruct((NW * B,), jnp.float32),
            jax.ShapeDtypeStruct((NW * B,), jnp.int32),
        ],
        scratch_types=[
            pltpu.VMEM((TPW * TILE,), jnp.float32),
            pltpu.VMEM((8, CT * TILE), jnp.float32),
            pltpu.VMEM((8, CT * TILE), jnp.float32),
            pltpu.VMEM((B,), jnp.float32),
            pltpu.VMEM((B,), jnp.float32),
            pltpu.VMEM((B,), jnp.int32),
            pltpu.SemaphoreType.DMA,
            pltpu.SemaphoreType.DMA,
        ],
    )
    def sc_sampler(logits_hbm, temps_hbm, noise_hbm, pvals_hbm, pidx_hbm,
                   nbuf, lbufa, lbufb, tbuf, vbuf, ibuf, sema, semb):
        cid = lax.axis_index("c")
        sid = lax.axis_index("s")
        wid = sid * NUM_CORES + cid
        t0 = jnp.minimum(wid * STEP, NT - TPW)
        pltpu.sync_copy(temps_hbm, tbuf)
        iota = lax.iota(jnp.int32, LANES)
        neg = jnp.full((LANES,), NEG_INF, jnp.float32)
        zero = jnp.zeros((LANES,), jnp.int32)
        bufs = (lbufa, lbufb)
        sems = (sema, semb)

        def chunk_copy(rg, ci, parity):
            return pltpu.make_async_copy(
                logits_hbm.at[pl.ds(rg * 8, 8),
                              pl.ds((t0 + ci * CT) * TILE, CT * TILE)],
                bufs[parity], sems[parity])

        chunk_copy(0, NCH - 1, 0).start()
        chunk_copy(0, 0, 1).start()
        pltpu.sync_copy(noise_hbm.at[pl.ds(t0 * TILE, TPW * TILE)], nbuf)

        tg0 = tbuf[pl.ds(0, LANES)]
        tg1 = tbuf[pl.ds(LANES, LANES)]

        def proc(buf, chunkbase, avals, rv, ri):
            nbase = chunkbase * TILE
            basev = ((t0 + chunkbase) * TILE) + iota
            rv, ri = list(rv), list(ri)
            for rp in range(4):
                r0, r1 = 2 * rp, 2 * rp + 1
                a0, a1 = avals[r0], avals[r1]

                def step(ct, c, a0=a0, a1=a1, r0=r0, r1=r1):
                    vm0, vj0, vm1, vj1 = c
                    jd0 = zero + ct * 2
                    for tt in range(2):
                        jd = jd0 + tt
                        coff = (ct * 2 + tt) * TILE
                        g0 = g1 = None
                        for k in range(KPT):
                            off = coff + k * LANES
                            nv = nbuf[pl.ds(nbase + off, LANES)]
                            t0v = buf[r0, pl.ds(off, LANES)] - a0 * nv
                            t1v = buf[r1, pl.ds(off, LANES)] - a1 * nv
                            g0 = t0v if g0 is None else jnp.maximum(g0, t0v)
                            g1 = t1v if g1 is None else jnp.maximum(g1, t1v)
                        m0 = g0 > vm0
                        m1 = g1 > vm1
                        vm0 = jnp.where(m0, g0, vm0)
                        vj0 = jnp.where(m0, jd, vj0)
                        vm1 = jnp.where(m1, g1, vm1)
                        vj1 = jnp.where(m1, jd, vj1)
                    return (vm0, vj0, vm1, vj1)

                vm0, vj0, vm1, vj1 = lax.fori_loop(
                    0, CT // 2, step, (neg, zero, neg, zero))
                for r, vm, vj, a in ((r0, vm0, vj0, a0), (r1, vm1, vj1, a1)):
                    mv = jnp.max(vm)
                    hstar = jnp.min(jnp.where(vm == mv, vj,
                                              jnp.int32(INT_MAX)))
                    hoff = hstar * TILE
                    cand = None
                    for k in range(KPT):
                        nv = nbuf[pl.ds(nbase + hoff + k * LANES, LANES)]
                        lv = buf[r, pl.ds(hoff + k * LANES, LANES)]
                        kv = lv - a * nv
                        idxv = basev + (hoff + k * LANES)
                        ck = jnp.where(kv == mv, idxv, jnp.int32(INT_MAX))
                        cand = ck if cand is None else jnp.minimum(cand, ck)
                    bidx = jnp.min(cand)
                    take = (mv > rv[r]) | ((mv == rv[r]) & (bidx < ri[r]))
                    rv[r] = jnp.where(take, mv, rv[r])
                    ri[r] = jnp.where(take, bidx, ri[r])
            return rv, ri

        def rowgroup(rg, carry):
            resv0, resv1, resi0, resi1 = carry
            avals = []
            for r in range(8):
                b = rg * 8 + r
                tsel = jnp.where(b < 16, tg0, tg1)
                lm = iota == (b & (2 * LANES - 1)) % LANES
                t = jnp.max(jnp.where(lm, tsel, NEG_INF))
                avals.append(jnp.where(t >= EPS, t, 0.0))
            rv = [jnp.float32(NEG_INF)] * 8
            ri = [jnp.int32(0)] * 8
            chunk_copy(rg, NCH - 1, 0).wait()
            rv, ri = proc(lbufa, (NCH - 1) * CT, avals, rv, ri)
            chunk_copy(rg, 1, 0).start()

            def trips(tt, c):
                rvri = list(c)
                rv, ri = rvri[:8], rvri[8:]
                ce = tt * 2
                chunk_copy(rg, ce, 1).wait()
                rv, ri = proc(lbufb, ce * CT, avals, rv, ri)

                @pl.when(ce + 2 <= NCH - 2)
                def _sb():
                    chunk_copy(rg, ce + 2, 1).start()

                @pl.when((ce == NCH - 3) & (rg < 3))
                def _pb():
                    chunk_copy(rg + 1, 0, 1).start()

                co = tt * 2 + 1
                chunk_copy(rg, co, 0).wait()
                rv, ri = proc(lbufa, co * CT, avals, rv, ri)

                @pl.when(co + 2 <= NCH - 2)
                def _sa():
                    chunk_copy(rg, co + 2, 0).start()

                @pl.when((co == NCH - 2) & (rg < 3))
                def _pa():
                    chunk_copy(rg + 1, NCH - 1, 0).start()

                return tuple(rv + ri)

            assert (NCH - 1) % 2 == 0
            out = list(lax.fori_loop(0, (NCH - 1) // 2, trips,
                                     tuple(rv + ri)))
            rv, ri = out[:8], out[8:]

            for r in range(8):
                b = rg * 8 + r
                lm = iota == (b & (2 * LANES - 1)) % LANES
                lo = b < 16
                m0 = lm & lo
                m1 = lm & (~lo)
                resv0 = jnp.where(m0, rv[r], resv0)
                resi0 = jnp.where(m0, ri[r], resi0)
                resv1 = jnp.where(m1, rv[r], resv1)
                resi1 = jnp.where(m1, ri[r], resi1)
            return resv0, resv1, resi0, resi1

        resv0, resv1, resi0, resi1 = lax.fori_loop(
            0, 4, rowgroup, (neg, neg, zero, zero))
        vbuf[pl.ds(0, LANES)] = resv0
        vbuf[pl.ds(LANES, LANES)] = resv1
        ibuf[pl.ds(0, LANES)] = resi0
        ibuf[pl.ds(LANES, LANES)] = resi1
        pltpu.sync_copy(vbuf, pvals_hbm.at[pl.ds(wid * B, B)])
        pltpu.sync_copy(ibuf, pidx_hbm.at[pl.ds(wid * B, B)])

    return sc_sampler, NT * TILE


def kernel(logits, temperatures, exponential):
    B, V = logits.shape
    sampler, vcov = _build_sc_sampler(B, V)
    noise = _tc_log_noise(exponential)
    pv, pi = sampler(logits, temperatures, noise)
    pv = pv.reshape(NW, B)
    pi = pi.reshape(NW, B)
    if vcov < V:
        a = jnp.where(temperatures >= EPS, temperatures, 0.0)
        tk = logits[:, vcov:] - a[:, None] * noise[vcov:]
        tv = jnp.max(tk, axis=-1)
        ti = (vcov + jnp.argmax(tk, axis=-1)).astype(jnp.int32)
        pv = jnp.concatenate([pv, tv[None, :]], axis=0)
        pi = jnp.concatenate([pi, ti[None, :]], axis=0)
    w = jnp.argmax(pv, axis=0)
    out = jnp.take_along_axis(pi, w[None, :], axis=0)[0]
    return out.astype(jnp.int32)


# speedup vs baseline: 1.1033x; 1.1033x over previous
import functools

import jax
import jax.numpy as jnp
from jax import lax
from jax.experimental import pallas as pl
from jax.experimental.pallas import tpu as pltpu
from jax.experimental.pallas import tpu_sc as plsc

EPS = 1e-10
NUM_CORES = 2
NUM_SUBCORES = 16
LANES = 16
NW = NUM_CORES * NUM_SUBCORES
TILE = 128
KPT = TILE // LANES
NEG_INF = float("-inf")
INT_MAX = 2147483647

def _tc_log_noise(exponential):
    V = exponential.shape[1]
    C = 131072
    grid = -(-V // C)

    def body(e_ref, n_ref):
        n_ref[...] = jnp.log(jnp.maximum(e_ref[0, :], EPS))

    return pl.pallas_call(
        body,
        grid=(grid,),
        in_specs=[pl.BlockSpec((1, C), lambda i: (0, i))],
        out_specs=pl.BlockSpec((C,), lambda i: (i,)),
        out_shape=jax.ShapeDtypeStruct((V,), jnp.float32),
    )(exponential)


def _merge(cand):
    while len(cand) > 1:
        nxt = []
        for i in range(0, len(cand) - 1, 2):
            av, ai = cand[i]
            bv, bi = cand[i + 1]
            takeb = (bv > av) | ((bv == av) & (bi < ai))
            nxt.append((jnp.where(takeb, bv, av), jnp.where(takeb, bi, ai)))
        if len(cand) % 2:
            nxt.append(cand[-1])
        cand = nxt
    return cand[0]


@functools.lru_cache(maxsize=None)
def _build_sc_sampler(B, V):
    assert B == 32
    NT = V // TILE
    TPW = -(-NT // NW)
    CT = 35
    TPW = -(-TPW // CT) * CT
    NCH = TPW // CT
    assert NCH >= 3 and TPW <= NT
    STEP = -(-(NT - TPW) // (NW - 1))
    assert STEP <= TPW and (NW - 2) * STEP + TPW >= NT - TPW

    mesh = plsc.VectorSubcoreMesh(
        core_axis_name="c", subcore_axis_name="s",
        num_cores=NUM_CORES, num_subcores=NUM_SUBCORES)

    @functools.partial(
        pl.kernel,
        mesh=mesh,
        compiler_params=pltpu.CompilerParams(
            needs_layout_passes=False, use_tc_tiling_on_sc=True),
        out_type=[
            jax.ShapeDtypeStruct((NW * B,), jnp.float32),
            jax.ShapeDtypeStruct((NW * B,), jnp.int32),
        ],
        scratch_types=[
            pltpu.VMEM((TPW * TILE,), jnp.float32),
            pltpu.VMEM((8, CT * TILE), jnp.float32),
            pltpu.VMEM((8, CT * TILE), jnp.float32),
            pltpu.VMEM((B,), jnp.float32),
            pltpu.VMEM((B,), jnp.float32),
            pltpu.VMEM((B,), jnp.int32),
            pltpu.SemaphoreType.DMA,
            pltpu.SemaphoreType.DMA,
        ],
    )
    def sc_sampler(logits_hbm, temps_hbm, noise_hbm, pvals_hbm, pidx_hbm,
                   nbuf, lbufa, lbufb, tbuf, vbuf, ibuf, sema, semb):
        cid = lax.axis_index("c")
        sid = lax.axis_index("s")
        wid = sid * NUM_CORES + cid
        t0 = jnp.minimum(wid * STEP, NT - TPW)
        pltpu.sync_copy(temps_hbm, tbuf)
        iota = lax.iota(jnp.int32, LANES)
        neg = jnp.full((LANES,), NEG_INF, jnp.float32)
        zero = jnp.zeros((LANES,), jnp.int32)
        bufs = (lbufa, lbufb)
        sems = (sema, semb)

        def chunk_copy(rg, ci, parity):
            return pltpu.make_async_copy(
                logits_hbm.at[pl.ds(rg * 8, 8),
                              pl.ds((t0 + ci * CT) * TILE, CT * TILE)],
                bufs[parity], sems[parity])

        chunk_copy(0, NCH - 1, 0).start()
        chunk_copy(0, 0, 1).start()
        pltpu.sync_copy(noise_hbm.at[pl.ds(t0 * TILE, TPW * TILE)], nbuf)

        tg0 = tbuf[pl.ds(0, LANES)]
        tg1 = tbuf[pl.ds(LANES, LANES)]

        def proc(buf, chunkbase, avals, rv, ri):
            nbase = chunkbase * TILE
            basev = ((t0 + chunkbase) * TILE) + iota
            rv, ri = list(rv), list(ri)
            for rp in range(4):
                r0, r1 = 2 * rp, 2 * rp + 1
                a0, a1 = avals[r0], avals[r1]

                def step(ct, c, a0=a0, a1=a1, r0=r0, r1=r1):
                    vm0, vj0, vm1, vj1 = c
                    jd = zero + ct
                    coff = ct * TILE
                    g0 = g1 = None
                    for k in range(KPT):
                        off = coff + k * LANES
                        nv = nbuf[pl.ds(nbase + off, LANES)]
                        t0v = buf[r0, pl.ds(off, LANES)] - a0 * nv
                        t1v = buf[r1, pl.ds(off, LANES)] - a1 * nv
                        g0 = t0v if g0 is None else jnp.maximum(g0, t0v)
                        g1 = t1v if g1 is None else jnp.maximum(g1, t1v)
                    m0 = g0 > vm0
                    m1 = g1 > vm1
                    return (jnp.where(m0, g0, vm0), jnp.where(m0, jd, vj0),
                            jnp.where(m1, g1, vm1), jnp.where(m1, jd, vj1))

                vm0, vj0, vm1, vj1 = lax.fori_loop(
                    0, CT, step, (neg, zero, neg, zero))
                for r, vm, vj, a in ((r0, vm0, vj0, a0), (r1, vm1, vj1, a1)):
                    mv = jnp.max(vm)
                    hstar = jnp.min(jnp.where(vm == mv, vj,
                                              jnp.int32(INT_MAX)))
                    hoff = hstar * TILE
                    cand = None
                    for k in range(KPT):
                        nv = nbuf[pl.ds(nbase + hoff + k * LANES, LANES)]
                        lv = buf[r, pl.ds(hoff + k * LANES, LANES)]
                        kv = lv - a * nv
                        idxv = basev + (hoff + k * LANES)
                        ck = jnp.where(kv == mv, idxv, jnp.int32(INT_MAX))
                        cand = ck if cand is None else jnp.minimum(cand, ck)
                    bidx = jnp.min(cand)
                    take = (mv > rv[r]) | ((mv == rv[r]) & (bidx < ri[r]))
                    rv[r] = jnp.where(take, mv, rv[r])
                    ri[r] = jnp.where(take, bidx, ri[r])
            return rv, ri

        def rowgroup(rg, carry):
            resv0, resv1, resi0, resi1 = carry
            avals = []
            for r in range(8):
                b = rg * 8 + r
                tsel = jnp.where(b < 16, tg0, tg1)
                lm = iota == (b & (2 * LANES - 1)) % LANES
                t = jnp.max(jnp.where(lm, tsel, NEG_INF))
                avals.append(jnp.where(t >= EPS, t, 0.0))
            rv = [jnp.float32(NEG_INF)] * 8
            ri = [jnp.int32(0)] * 8
            chunk_copy(rg, NCH - 1, 0).wait()
            rv, ri = proc(lbufa, (NCH - 1) * CT, avals, rv, ri)
            chunk_copy(rg, 1, 0).start()

            def trips(tt, c):
                rvri = list(c)
                rv, ri = rvri[:8], rvri[8:]
                ce = tt * 2
                chunk_copy(rg, ce, 1).wait()
                rv, ri = proc(lbufb, ce * CT, avals, rv, ri)

                @pl.when(ce + 2 <= NCH - 2)
                def _sb():
                    chunk_copy(rg, ce + 2, 1).start()

                @pl.when((ce == NCH - 3) & (rg < 3))
                def _pb():
                    chunk_copy(rg + 1, 0, 1).start()

                co = tt * 2 + 1
                chunk_copy(rg, co, 0).wait()
                rv, ri = proc(lbufa, co * CT, avals, rv, ri)

                @pl.when(co + 2 <= NCH - 2)
                def _sa():
                    chunk_copy(rg, co + 2, 0).start()

                @pl.when((co == NCH - 2) & (rg < 3))
                def _pa():
                    chunk_copy(rg + 1, NCH - 1, 0).start()

                return tuple(rv + ri)

            assert (NCH - 1) % 2 == 0
            out = list(lax.fori_loop(0, (NCH - 1) // 2, trips,
                                     tuple(rv + ri)))
            rv, ri = out[:8], out[8:]

            for r in range(8):
                b = rg * 8 + r
                lm = iota == (b & (2 * LANES - 1)) % LANES
                lo = b < 16
                m0 = lm & lo
                m1 = lm & (~lo)
                resv0 = jnp.where(m0, rv[r], resv0)
                resi0 = jnp.where(m0, ri[r], resi0)
                resv1 = jnp.where(m1, rv[r], resv1)
                resi1 = jnp.where(m1, ri[r], resi1)
            return resv0, resv1, resi0, resi1

        resv0, resv1, resi0, resi1 = lax.fori_loop(
            0, 4, rowgroup, (neg, neg, zero, zero))
        vbuf[pl.ds(0, LANES)] = resv0
        vbuf[pl.ds(LANES, LANES)] = resv1
        ibuf[pl.ds(0, LANES)] = resi0
        ibuf[pl.ds(LANES, LANES)] = resi1
        pltpu.sync_copy(vbuf, pvals_hbm.at[pl.ds(wid * B, B)])
        pltpu.sync_copy(ibuf, pidx_hbm.at[pl.ds(wid * B, B)])

    return sc_sampler, NT * TILE


def kernel(logits, temperatures, exponential):
    B, V = logits.shape
    sampler, vcov = _build_sc_sampler(B, V)
    noise = _tc_log_noise(exponential)
    pv, pi = sampler(logits, temperatures, noise)
    pv = pv.reshape(NW, B)
    pi = pi.reshape(NW, B)
    if vcov < V:
        a = jnp.where(temperatures >= EPS, temperatures, 0.0)
        tk = logits[:, vcov:] - a[:, None] * noise[vcov:]
        tv = jnp.max(tk, axis=-1)
        ti = (vcov + jnp.argmax(tk, axis=-1)).astype(jnp.int32)
        pv = jnp.concatenate([pv, tv[None, :]], axis=0)
        pi = jnp.concatenate([pi, ti[None, :]], axis=0)
    w = jnp.argmax(pv, axis=0)
    out = jnp.take_along_axis(pi, w[None, :], axis=0)[0]
    return out.astype(jnp.int32)
